# baseline (speedup 1.0000x reference)
import functools

import jax
import jax.numpy as jnp
from jax import lax
from jax.experimental import pallas as pl
from jax.experimental.pallas import tpu as pltpu
from jax.experimental.pallas import tpu_sc as plsc

_S, _Q, _E, _D = 20, 500, 16000, 128
_QP = 512
_CHUNK = 64
_NCQ = _QP // _CHUNK
_NTASK = _S * _NCQ
_NW = 32
_TPW = _NTASK // _NW

_mesh = plsc.VectorSubcoreMesh(core_axis_name="c", subcore_axis_name="s")


@functools.partial(
    pl.kernel,
    out_type=jax.ShapeDtypeStruct((_S, _QP, _QP), jnp.float32),
    mesh=_mesh,
    scratch_types=[
        pltpu.VMEM((2, _CHUNK, _QP), jnp.float32),
        pltpu.VMEM((2, 2, _E), jnp.int32),
        pltpu.SemaphoreType.DMA,
        pltpu.SemaphoreType.DMA,
        pltpu.SemaphoreType.DMA,
        pltpu.SemaphoreType.DMA,
    ],
    compiler_params=pltpu.CompilerParams(needs_layout_passes=False),
)
def _build_counts(edges_hbm, out_hbm, acc, ebuf, sem0, sem1, wsem0, wsem1):
    wid = lax.axis_index("s") * 2 + lax.axis_index("c")
    ones = jnp.ones((16,), jnp.float32)
    zeros = jnp.zeros((16,), jnp.float32)
    sems = [sem0, sem1]
    wsems = [wsem0, wsem1]

    def task(k):
        t = wid + k * _NW
        return t // _NCQ, (t % _NCQ) * _CHUNK

    s0, _ = task(0)
    pltpu.async_copy(edges_hbm.at[s0], ebuf.at[0], sems[0])

    for k in range(_TPW):
        b = k % 2
        s, c0 = task(k)
        with jax.named_scope("edge_wait"):
            pltpu.make_async_copy(edges_hbm.at[s], ebuf.at[b], sems[b]).wait()
        if k + 1 < _TPW:
            s2, _ = task(k + 1)
            pltpu.async_copy(edges_hbm.at[s2], ebuf.at[1 - b], sems[1 - b])
        if k >= 2:
            sp, cp = task(k - 2)
            with jax.named_scope("wb_wait"):
                pltpu.make_async_copy(
                    acc.at[b], out_hbm.at[sp, pl.ds(cp, _CHUNK), :],
                    wsems[b]).wait()

        with jax.named_scope("zero"):
            @plsc.parallel_loop(0, _CHUNK, unroll=2)
            def _zero(i):
                for j in range(_QP // 16):
                    acc[b, i, pl.ds(j * 16, 16)] = zeros

        with jax.named_scope("scan"):
            @plsc.parallel_loop(0, _E // 16, unroll=8)
            def _edges(g):
                r = ebuf[b, 0, pl.ds(g * 16, 16)]
                c = ebuf[b, 1, pl.ds(g * 16, 16)]
                cl = c - c0
                valid = (cl >= 0) & (cl < _CHUNK)
                plsc.addupdate_scatter(acc.at[b], [cl, r], ones, mask=valid)

        with jax.named_scope("writeback"):
            pltpu.async_copy(
                acc.at[b], out_hbm.at[s, pl.ds(c0, _CHUNK), :], wsems[b])

    for k in range(_TPW - 2, _TPW):
        b = k % 2
        s, c0 = task(k)
        pltpu.make_async_copy(
            acc.at[b], out_hbm.at[s, pl.ds(c0, _CHUNK), :], wsems[b]).wait()


_GPB = 4


def _gcn_body(c_ref, emb_ref, w1_ref, b1_ref, w2_ref, b2_ref, out_ref):
    h0 = jnp.dot(emb_ref[...], w1_ref[...], preferred_element_type=jnp.float32)
    for g in range(_GPB):
        Cz = c_ref[g]
        C = Cz[:_Q, :_Q].astype(jnp.bfloat16)
        deg = jnp.sum(Cz, axis=1, keepdims=True)[:_Q] + 1.0
        dinv = 1.0 / jnp.sqrt(deg)
        h = h0 * dinv
        y = (jnp.dot(C, h.astype(jnp.bfloat16),
                     preferred_element_type=jnp.float32) + h) * dinv
        x1 = jnp.maximum(y + b1_ref[...], 0.0)
        z = jnp.dot(x1, w2_ref[...], preferred_element_type=jnp.float32) * dinv
        y2 = (jnp.dot(C, z.astype(jnp.bfloat16),
                      preferred_element_type=jnp.float32) + z) * dinv
        out_ref[pl.ds(g * _Q, _Q)] = jnp.maximum(y2 + b2_ref[...], 0.0)


_gcn_tc = pl.pallas_call(
    _gcn_body,
    grid=(_S // _GPB,),
    in_specs=[
        pl.BlockSpec((_GPB, _QP, _QP), lambda s: (s, 0, 0)),
        pl.BlockSpec((_Q, _D), lambda s: (0, 0)),
        pl.BlockSpec((_D, _D), lambda s: (0, 0)),
        pl.BlockSpec((1, _D), lambda s: (0, 0)),
        pl.BlockSpec((_D, _D), lambda s: (0, 0)),
        pl.BlockSpec((1, _D), lambda s: (0, 0)),
    ],
    out_specs=pl.BlockSpec((_GPB * _Q, _D), lambda s: (s, 0)),
    out_shape=jax.ShapeDtypeStruct((_S * _Q, _D), jnp.float32),
)


def kernel(slice_matrices, qubit_embs, W1, b1, W2, b2):
    edges = slice_matrices.astype(jnp.int32)
    counts = _build_counts(edges)
    return _gcn_tc(counts, qubit_embs, W1, b1.reshape(1, _D), W2, b2.reshape(1, _D))


# speedup vs baseline: 1.0167x; 1.0167x over previous
import functools

import jax
import jax.numpy as jnp
from jax import lax
from jax.experimental import pallas as pl
from jax.experimental.pallas import tpu as pltpu
from jax.experimental.pallas import tpu_sc as plsc

_S, _Q, _E, _D = 20, 500, 16000, 128
_QP = 512
_CHUNK = 128
_NCQ = _QP // _CHUNK
_NTASK = _S * _NCQ
_NW = 32
_TPW = -(-_NTASK // _NW)

_mesh = plsc.VectorSubcoreMesh(core_axis_name="c", subcore_axis_name="s")


@functools.partial(
    pl.kernel,
    out_type=jax.ShapeDtypeStruct((_S, _QP, _QP), jnp.float32),
    mesh=_mesh,
    scratch_types=[
        pltpu.VMEM((_CHUNK, _QP), jnp.float32),
        pltpu.VMEM((2, 2, _E), jnp.int32),
        pltpu.SemaphoreType.DMA,
        pltpu.SemaphoreType.DMA,
    ],
    compiler_params=pltpu.CompilerParams(needs_layout_passes=False),
)
def _build_counts(edges_hbm, out_hbm, acc, ebuf, sem0, sem1):
    wid = lax.axis_index("s") * 2 + lax.axis_index("c")
    ones = jnp.ones((16,), jnp.float32)
    zeros = jnp.zeros((16,), jnp.float32)
    sems = [sem0, sem1]

    pltpu.async_copy(edges_hbm.at[wid // _NCQ], ebuf.at[0], sems[0])

    for k in range(_TPW):
        t = wid + k * _NW

        @pl.when(t < _NTASK)
        def _():
            s = t // _NCQ
            c0 = (t % _NCQ) * _CHUNK
            with jax.named_scope("edge_wait"):
                pltpu.make_async_copy(
                    edges_hbm.at[s], ebuf.at[k % 2], sems[k % 2]).wait()
            if k + 1 < _TPW:
                t2 = t + _NW

                @pl.when(t2 < _NTASK)
                def _prefetch():
                    pltpu.async_copy(
                        edges_hbm.at[t2 // _NCQ], ebuf.at[(k + 1) % 2],
                        sems[(k + 1) % 2])

            with jax.named_scope("zero"):
                @plsc.parallel_loop(0, _CHUNK, unroll=2)
                def _zero(i):
                    for j in range(_QP // 16):
                        acc[i, pl.ds(j * 16, 16)] = zeros

            with jax.named_scope("scan"):
                @plsc.parallel_loop(0, _E // 16, unroll=4)
                def _edges(g):
                    r = ebuf[k % 2, 0, pl.ds(g * 16, 16)]
                    c = ebuf[k % 2, 1, pl.ds(g * 16, 16)]
                    cl = c - c0
                    valid = (cl >= 0) & (cl < _CHUNK)
                    plsc.addupdate_scatter(acc, [cl, r], ones, mask=valid)

            with jax.named_scope("writeback"):
                pltpu.sync_copy(acc, out_hbm.at[s, pl.ds(c0, _CHUNK), :])


_GPB = 4


def _gcn_body(c_ref, emb_ref, w1_ref, b1_ref, w2_ref, b2_ref, out_ref):
    h0 = jnp.dot(emb_ref[...], w1_ref[...], preferred_element_type=jnp.float32)
    for g in range(_GPB):
        Cz = c_ref[g]
        C = Cz[:_Q, :_Q].astype(jnp.bfloat16)
        deg = jnp.sum(Cz, axis=1, keepdims=True)[:_Q] + 1.0
        dinv = 1.0 / jnp.sqrt(deg)
        h = h0 * dinv
        y = (jnp.dot(C, h.astype(jnp.bfloat16),
                     preferred_element_type=jnp.float32) + h) * dinv
        x1 = jnp.maximum(y + b1_ref[...], 0.0)
        z = jnp.dot(x1, w2_ref[...], preferred_element_type=jnp.float32) * dinv
        y2 = (jnp.dot(C, z.astype(jnp.bfloat16),
                      preferred_element_type=jnp.float32) + z) * dinv
        out_ref[pl.ds(g * _Q, _Q)] = jnp.maximum(y2 + b2_ref[...], 0.0)


_gcn_tc = pl.pallas_call(
    _gcn_body,
    grid=(_S // _GPB,),
    in_specs=[
        pl.BlockSpec((_GPB, _QP, _QP), lambda s: (s, 0, 0)),
        pl.BlockSpec((_Q, _D), lambda s: (0, 0)),
        pl.BlockSpec((_D, _D), lambda s: (0, 0)),
        pl.BlockSpec((1, _D), lambda s: (0, 0)),
        pl.BlockSpec((_D, _D), lambda s: (0, 0)),
        pl.BlockSpec((1, _D), lambda s: (0, 0)),
    ],
    out_specs=pl.BlockSpec((_GPB * _Q, _D), lambda s: (s, 0)),
    out_shape=jax.ShapeDtypeStruct((_S * _Q, _D), jnp.float32),
)


def kernel(slice_matrices, qubit_embs, W1, b1, W2, b2):
    edges = slice_matrices.astype(jnp.int32)
    counts = _build_counts(edges)
    return _gcn_tc(counts, qubit_embs, W1, b1.reshape(1, _D), W2, b2.reshape(1, _D))


# speedup vs baseline: 1.0179x; 1.0012x over previous
import functools

import jax
import jax.numpy as jnp
from jax import lax
from jax.experimental import pallas as pl
from jax.experimental.pallas import tpu as pltpu
from jax.experimental.pallas import tpu_sc as plsc

_S, _Q, _E, _D = 20, 500, 16000, 128
_QP = 512
_CHUNK = 128
_NCQ = _QP // _CHUNK
_NTASK = _S * _NCQ
_NW = 32
_TPW = -(-_NTASK // _NW)

_mesh = plsc.VectorSubcoreMesh(core_axis_name="c", subcore_axis_name="s")


@functools.partial(
    pl.kernel,
    out_type=jax.ShapeDtypeStruct((_S, _QP, _QP), jnp.float32),
    mesh=_mesh,
    scratch_types=[
        pltpu.VMEM((_CHUNK, _QP), jnp.float32),
        pltpu.VMEM((2, 2, _E), jnp.int32),
        pltpu.SemaphoreType.DMA,
        pltpu.SemaphoreType.DMA,
    ],
    compiler_params=pltpu.CompilerParams(needs_layout_passes=False),
)
def _build_counts(edges_hbm, out_hbm, acc, ebuf, sem0, sem1):
    wid = lax.axis_index("s") * 2 + lax.axis_index("c")
    ones = jnp.ones((16,), jnp.float32)
    zeros = jnp.zeros((16,), jnp.float32)
    sems = [sem0, sem1]

    pltpu.async_copy(edges_hbm.at[wid // _NCQ], ebuf.at[0], sems[0])

    for k in range(_TPW):
        t = wid + k * _NW

        @pl.when(t < _NTASK)
        def _():
            s = t // _NCQ
            c0 = (t % _NCQ) * _CHUNK
            pltpu.make_async_copy(
                edges_hbm.at[s], ebuf.at[k % 2], sems[k % 2]).wait()
            if k + 1 < _TPW:
                t2 = t + _NW

                @pl.when(t2 < _NTASK)
                def _prefetch():
                    pltpu.async_copy(
                        edges_hbm.at[t2 // _NCQ], ebuf.at[(k + 1) % 2],
                        sems[(k + 1) % 2])

            @plsc.parallel_loop(0, _CHUNK, unroll=2)
            def _zero(i):
                for j in range(_QP // 16):
                    acc[i, pl.ds(j * 16, 16)] = zeros

            @plsc.parallel_loop(0, _E // 16, unroll=4)
            def _edges(g):
                r = ebuf[k % 2, 0, pl.ds(g * 16, 16)]
                c = ebuf[k % 2, 1, pl.ds(g * 16, 16)]
                cl = c - c0
                valid = (cl >= 0) & (cl < _CHUNK)
                plsc.addupdate_scatter(acc, [cl, r], ones, mask=valid)

            pltpu.sync_copy(acc, out_hbm.at[s, pl.ds(c0, _CHUNK), :])


_GPB = 4


def _gcn_body(c_ref, emb_ref, w1_ref, b1_ref, w2_ref, b2_ref, out_ref):
    h0 = jnp.dot(emb_ref[...], w1_ref[...], preferred_element_type=jnp.float32)
    for g in range(_GPB):
        Cz = c_ref[g]
        C = Cz[:_Q, :_Q].astype(jnp.bfloat16)
        deg = jnp.sum(Cz, axis=1, keepdims=True)[:_Q] + 1.0
        dinv = 1.0 / jnp.sqrt(deg)
        h = h0 * dinv
        y = (jnp.dot(C, h.astype(jnp.bfloat16),
                     preferred_element_type=jnp.float32) + h) * dinv
        x1 = jnp.maximum(y + b1_ref[...], 0.0)
        z = jnp.dot(x1, w2_ref[...], preferred_element_type=jnp.float32) * dinv
        y2 = (jnp.dot(C, z.astype(jnp.bfloat16),
                      preferred_element_type=jnp.float32) + z) * dinv
        out_ref[pl.ds(g * _Q, _Q)] = jnp.maximum(y2 + b2_ref[...], 0.0)


_gcn_tc = pl.pallas_call(
    _gcn_body,
    grid=(_S // _GPB,),
    in_specs=[
        pl.BlockSpec((_GPB, _QP, _QP), lambda s: (s, 0, 0)),
        pl.BlockSpec((_Q, _D), lambda s: (0, 0)),
        pl.BlockSpec((_D, _D), lambda s: (0, 0)),
        pl.BlockSpec((1, _D), lambda s: (0, 0)),
        pl.BlockSpec((_D, _D), lambda s: (0, 0)),
        pl.BlockSpec((1, _D), lambda s: (0, 0)),
    ],
    out_specs=pl.BlockSpec((_GPB * _Q, _D), lambda s: (s, 0)),
    out_shape=jax.ShapeDtypeStruct((_S * _Q, _D), jnp.float32),
)


def kernel(slice_matrices, qubit_embs, W1, b1, W2, b2):
    edges = slice_matrices.astype(jnp.int32)
    counts = _build_counts(edges)
    return _gcn_tc(counts, qubit_embs, W1, b1.reshape(1, _D), W2, b2.reshape(1, _D))
